# initial kernel scaffold (unmeasured)
import jax
import jax.numpy as jnp
from jax import lax
from jax.experimental import pallas as pl
from jax.experimental.pallas import tpu as pltpu

N_DEV = 4
SQ = 256
D = 1024
HQ = 8
DH = 128
SCALE = 0.08838834764831843


def kernel(x, Wq, Wo, K_ext, V_ext):
    x2 = x.reshape(SQ, D)
    k2 = K_ext.reshape(-1, HQ * DH)
    v2 = V_ext.reshape(-1, HQ * DH)

    def body(
        x_ref,
        wq_ref,
        wo_ref,
        k_ref,
        v_ref,
        out_ref,
        send_o,
        send_l,
        recv_o,
        recv_l,
        send_sems_o,
        send_sems_l,
        recv_sems_o,
        recv_sems_l,
    ):
        my_i = lax.axis_index("i")

        barrier = pltpu.get_barrier_semaphore()
        for d in (1, 2, 3):
            pl.semaphore_signal(
                barrier,
                inc=1,
                device_id=((my_i + d) % N_DEV,),
                device_id_type=pl.DeviceIdType.MESH,
            )
        pl.semaphore_wait(barrier, 3)

        q = jnp.dot(x_ref[...], wq_ref[...], preferred_element_type=jnp.float32)
        for h in range(HQ):
            cols = pl.ds(h * DH, DH)
            s = lax.dot_general(
                q[:, h * DH:(h + 1) * DH],
                k_ref[:, cols],
                (((1,), (1,)), ((), ())),
                preferred_element_type=jnp.float32,
            )
            p = jnp.exp(s * SCALE)
            send_l[:, h:h + 1] = jnp.sum(p, axis=1, keepdims=True)
            send_o[:, cols] = jnp.dot(
                p, v_ref[:, cols], preferred_element_type=jnp.float32
            )

        rdmas = []
        for d in (1, 2, 3):
            peer = (my_i + d) % N_DEV
            ro = pltpu.make_async_remote_copy(
                src_ref=send_o,
                dst_ref=recv_o.at[d - 1],
                send_sem=send_sems_o.at[d - 1],
                recv_sem=recv_sems_o.at[d - 1],
                device_id=(peer,),
                device_id_type=pl.DeviceIdType.MESH,
            )
            ro.start()
            rl = pltpu.make_async_remote_copy(
                src_ref=send_l,
                dst_ref=recv_l.at[d - 1],
                send_sem=send_sems_l.at[d - 1],
                recv_sem=recv_sems_l.at[d - 1],
                device_id=(peer,),
                device_id_type=pl.DeviceIdType.MESH,
            )
            rl.start()
            rdmas.append((ro, rl))

        for ro, rl in rdmas:
            ro.wait()
            rl.wait()

        o_tot = send_o[...] + recv_o[0] + recv_o[1] + recv_o[2]
        l_tot = send_l[...] + recv_l[0] + recv_l[1] + recv_l[2]
        attn = jnp.concatenate(
            [
                o_tot[:, h * DH:(h + 1) * DH] / l_tot[:, h:h + 1]
                for h in range(HQ)
            ],
            axis=1,
        )
        out_ref[...] = jnp.dot(
            attn, wo_ref[...], preferred_element_type=jnp.float32
        )

    out = pl.pallas_call(
        body,
        out_shape=jax.ShapeDtypeStruct((SQ, D), jnp.float32),
        in_specs=[pl.BlockSpec(memory_space=pltpu.VMEM)] * 5,
        out_specs=pl.BlockSpec(memory_space=pltpu.VMEM),
        scratch_shapes=[
            pltpu.VMEM((SQ, D), jnp.float32),
            pltpu.VMEM((SQ, HQ), jnp.float32),
            pltpu.VMEM((N_DEV - 1, SQ, D), jnp.float32),
            pltpu.VMEM((N_DEV - 1, SQ, HQ), jnp.float32),
            pltpu.SemaphoreType.DMA((N_DEV - 1,)),
            pltpu.SemaphoreType.DMA((N_DEV - 1,)),
            pltpu.SemaphoreType.DMA((N_DEV - 1,)),
            pltpu.SemaphoreType.DMA((N_DEV - 1,)),
        ],
        compiler_params=pltpu.CompilerParams(collective_id=0),
    )(x2, Wq, Wo, k2, v2)
    return out.reshape(1, SQ, D)


# baseline (device time: 86658 ns/iter reference)
import jax
import jax.numpy as jnp
from jax import lax
from jax.experimental import pallas as pl
from jax.experimental.pallas import tpu as pltpu

N_DEV = 4
SQ = 256
D = 1024
HQ = 8
DH = 128
SCALE = 0.08838834764831843


def kernel(x, Wq, Wo, K_ext, V_ext):
    x2 = x.reshape(SQ, D)
    k2 = K_ext.reshape(-1, HQ * DH)
    v2 = V_ext.reshape(-1, HQ * DH)

    def body(
        x_ref,
        wq_ref,
        wo_ref,
        k_ref,
        v_ref,
        out_ref,
        send_o,
        send_l,
        recv_o,
        recv_l,
        send_sems_o,
        send_sems_l,
        recv_sems_o,
        recv_sems_l,
    ):
        my_i = lax.axis_index("i")

        barrier = pltpu.get_barrier_semaphore()
        for d in (1, 2, 3):
            pl.semaphore_signal(
                barrier,
                inc=1,
                device_id=((my_i + d) % N_DEV,),
                device_id_type=pl.DeviceIdType.MESH,
            )
        pl.semaphore_wait(barrier, 3)

        q = jnp.dot(x_ref[...], wq_ref[...], preferred_element_type=jnp.float32)
        for h in range(HQ):
            cols = pl.ds(h * DH, DH)
            s = lax.dot_general(
                q[:, h * DH:(h + 1) * DH],
                k_ref[:, cols],
                (((1,), (1,)), ((), ())),
                preferred_element_type=jnp.float32,
            )
            p = jnp.exp(s * SCALE)
            send_l[:, h:h + 1] = jnp.sum(p, axis=1, keepdims=True)
            send_o[:, cols] = jnp.dot(
                p, v_ref[:, cols], preferred_element_type=jnp.float32
            )

        rdmas = []
        for d in (1, 2, 3):
            peer = (my_i + d) % N_DEV
            ro = pltpu.make_async_remote_copy(
                src_ref=send_o,
                dst_ref=recv_o.at[d - 1],
                send_sem=send_sems_o.at[d - 1],
                recv_sem=recv_sems_o.at[d - 1],
                device_id=(peer,),
                device_id_type=pl.DeviceIdType.MESH,
            )
            ro.start()
            rl = pltpu.make_async_remote_copy(
                src_ref=send_l,
                dst_ref=recv_l.at[d - 1],
                send_sem=send_sems_l.at[d - 1],
                recv_sem=recv_sems_l.at[d - 1],
                device_id=(peer,),
                device_id_type=pl.DeviceIdType.MESH,
            )
            rl.start()
            rdmas.append((ro, rl))

        for ro, rl in rdmas:
            ro.wait()
            rl.wait()

        o_tot = send_o[...] + recv_o[0] + recv_o[1] + recv_o[2]
        l_tot = send_l[...] + recv_l[0] + recv_l[1] + recv_l[2]
        attn = jnp.concatenate(
            [
                o_tot[:, h * DH:(h + 1) * DH] / l_tot[:, h:h + 1]
                for h in range(HQ)
            ],
            axis=1,
        )
        out_ref[...] = jnp.dot(
            attn, wo_ref[...], preferred_element_type=jnp.float32
        )

    out = pl.pallas_call(
        body,
        out_shape=jax.ShapeDtypeStruct((SQ, D), jnp.float32),
        in_specs=[pl.BlockSpec(memory_space=pltpu.VMEM)] * 5,
        out_specs=pl.BlockSpec(memory_space=pltpu.VMEM),
        scratch_shapes=[
            pltpu.VMEM((SQ, D), jnp.float32),
            pltpu.VMEM((SQ, HQ), jnp.float32),
            pltpu.VMEM((N_DEV - 1, SQ, D), jnp.float32),
            pltpu.VMEM((N_DEV - 1, SQ, HQ), jnp.float32),
            pltpu.SemaphoreType.DMA((N_DEV - 1,)),
            pltpu.SemaphoreType.DMA((N_DEV - 1,)),
            pltpu.SemaphoreType.DMA((N_DEV - 1,)),
            pltpu.SemaphoreType.DMA((N_DEV - 1,)),
        ],
        compiler_params=pltpu.CompilerParams(
            collective_id=0,
            vmem_limit_bytes=100 * 1024 * 1024,
        ),
    )(x2, Wq, Wo, k2, v2)
    return out.reshape(1, SQ, D)


# device time: 62848 ns/iter; 1.3789x vs baseline; 1.3789x over previous
import jax
import jax.numpy as jnp
from jax import lax
from jax.experimental import pallas as pl
from jax.experimental.pallas import tpu as pltpu

N_DEV = 4
SQ = 256
D = 1024
HQ = 8
DH = 128
HALF = D // 2
SCALE = 0.08838834764831843


def kernel(x, Wq, Wo, K_ext, V_ext):
    bf16 = jnp.bfloat16
    x2 = x.reshape(SQ, D).astype(bf16)
    k2 = K_ext.reshape(-1, HQ * DH).astype(bf16)
    v2 = V_ext.reshape(-1, HQ * DH).astype(bf16)
    wq = Wq.astype(bf16)
    wo = Wo.astype(bf16)

    def body(
        x_ref,
        wq_ref,
        wo_ref,
        k_ref,
        v_ref,
        out_ref,
        send_o,
        send_l,
        recv_o,
        recv_l,
        send_sems_o,
        send_sems_l,
        recv_sems_o,
        recv_sems_l,
    ):
        my_i = lax.axis_index("i")

        barrier = pltpu.get_barrier_semaphore()
        for d in (1, 2, 3):
            pl.semaphore_signal(
                barrier,
                inc=1,
                device_id=((my_i + d) % N_DEV,),
                device_id_type=pl.DeviceIdType.MESH,
            )
        pl.semaphore_wait(barrier, 3)

        q = jnp.dot(x_ref[...], wq_ref[...], preferred_element_type=jnp.float32)
        qb = (q * SCALE).astype(jnp.bfloat16)

        rdmas = []

        def compute_half(half):
            for hh in range(HQ // 2):
                h = half * (HQ // 2) + hh
                cols = pl.ds(h * DH, DH)
                s = lax.dot_general(
                    qb[:, h * DH:(h + 1) * DH],
                    k_ref[:, cols],
                    (((1,), (1,)), ((), ())),
                    preferred_element_type=jnp.float32,
                )
                p = jnp.exp(s)
                send_l[:, h:h + 1] = jnp.sum(p, axis=1, keepdims=True)
                o = jnp.dot(
                    p.astype(jnp.bfloat16),
                    v_ref[:, cols],
                    preferred_element_type=jnp.float32,
                )
                send_o[half, :, pl.ds(hh * DH, DH)] = o.astype(jnp.bfloat16)

        def send_half(half, with_l):
            for d in (1, 2, 3):
                peer = (my_i + d) % N_DEV
                ro = pltpu.make_async_remote_copy(
                    src_ref=send_o.at[half],
                    dst_ref=recv_o.at[d - 1, half],
                    send_sem=send_sems_o.at[d - 1, half],
                    recv_sem=recv_sems_o.at[d - 1, half],
                    device_id=(peer,),
                    device_id_type=pl.DeviceIdType.MESH,
                )
                ro.start()
                rdmas.append(ro)
                if with_l:
                    rl = pltpu.make_async_remote_copy(
                        src_ref=send_l,
                        dst_ref=recv_l.at[d - 1],
                        send_sem=send_sems_l.at[d - 1],
                        recv_sem=recv_sems_l.at[d - 1],
                        device_id=(peer,),
                        device_id_type=pl.DeviceIdType.MESH,
                    )
                    rl.start()
                    rdmas.append(rl)

        compute_half(0)
        send_half(0, with_l=False)
        compute_half(1)
        send_half(1, with_l=True)

        for r in rdmas:
            r.wait()

        l_tot = send_l[...] + recv_l[0] + recv_l[1] + recv_l[2]
        cols_out = []
        for half in range(2):
            o_tot = (
                send_o[half].astype(jnp.float32)
                + recv_o[0, half].astype(jnp.float32)
                + recv_o[1, half].astype(jnp.float32)
                + recv_o[2, half].astype(jnp.float32)
            )
            for hh in range(HQ // 2):
                h = half * (HQ // 2) + hh
                cols_out.append(
                    (
                        o_tot[:, hh * DH:(hh + 1) * DH] / l_tot[:, h:h + 1]
                    ).astype(jnp.bfloat16)
                )
        attn = jnp.concatenate(cols_out, axis=1)
        out_ref[...] = jnp.dot(
            attn, wo_ref[...], preferred_element_type=jnp.float32
        )

    out = pl.pallas_call(
        body,
        out_shape=jax.ShapeDtypeStruct((SQ, D), jnp.float32),
        in_specs=[pl.BlockSpec(memory_space=pltpu.VMEM)] * 5,
        out_specs=pl.BlockSpec(memory_space=pltpu.VMEM),
        scratch_shapes=[
            pltpu.VMEM((2, SQ, HALF), jnp.bfloat16),
            pltpu.VMEM((SQ, HQ), jnp.float32),
            pltpu.VMEM((N_DEV - 1, 2, SQ, HALF), jnp.bfloat16),
            pltpu.VMEM((N_DEV - 1, SQ, HQ), jnp.float32),
            pltpu.SemaphoreType.DMA((N_DEV - 1, 2)),
            pltpu.SemaphoreType.DMA((N_DEV - 1,)),
            pltpu.SemaphoreType.DMA((N_DEV - 1, 2)),
            pltpu.SemaphoreType.DMA((N_DEV - 1,)),
        ],
        compiler_params=pltpu.CompilerParams(
            collective_id=0,
            vmem_limit_bytes=100 * 1024 * 1024,
        ),
    )(x2, wq, wo, k2, v2)
    return out.reshape(1, SQ, D)


# device time: 38436 ns/iter; 2.2546x vs baseline; 1.6351x over previous
import jax
import jax.numpy as jnp
from jax import lax
from jax.experimental import pallas as pl
from jax.experimental.pallas import tpu as pltpu

N_DEV = 4
SQ = 256
D = 1024
HQ = 8
DH = 128
HALF = D // 2
SCALE = 0.08838834764831843


def kernel(x, Wq, Wo, K_ext, V_ext):
    x2 = x.reshape(SQ, D)
    k3 = K_ext.reshape(-1, HQ, DH)
    v3 = V_ext.reshape(-1, HQ, DH)
    skv = k3.shape[0]

    def body(
        x_ref,
        wq_ref,
        wo_ref,
        k_hbm,
        v_hbm,
        out_ref,
        kbuf,
        vbuf,
        qb,
        send_o,
        send_l,
        recv_o,
        recv_l,
        kv_sems,
        send_sems_o,
        send_sems_l,
        recv_sems_o,
        recv_sems_l,
    ):
        my_i = lax.axis_index("i")

        def kv_copies(h):
            slot = h % 2
            return (
                pltpu.make_async_copy(
                    k_hbm.at[:, h, :], kbuf.at[slot], kv_sems.at[slot, 0]
                ),
                pltpu.make_async_copy(
                    v_hbm.at[:, h, :], vbuf.at[slot], kv_sems.at[slot, 1]
                ),
            )

        def peer_copies(half, with_l):
            copies = []
            for d in (1, 2, 3):
                peer = (my_i + d) % N_DEV
                copies.append(pltpu.make_async_remote_copy(
                    src_ref=send_o.at[half],
                    dst_ref=recv_o.at[d - 1, half],
                    send_sem=send_sems_o.at[d - 1, half],
                    recv_sem=recv_sems_o.at[d - 1, half],
                    device_id=(peer,),
                    device_id_type=pl.DeviceIdType.MESH,
                ))
                if with_l:
                    copies.append(pltpu.make_async_remote_copy(
                        src_ref=send_l,
                        dst_ref=recv_l.at[d - 1],
                        send_sem=send_sems_l.at[d - 1],
                        recv_sem=recv_sems_l.at[d - 1],
                        device_id=(peer,),
                        device_id_type=pl.DeviceIdType.MESH,
                    ))
            return copies

        barrier = pltpu.get_barrier_semaphore()
        for d in (1, 2, 3):
            pl.semaphore_signal(
                barrier,
                inc=1,
                device_id=((my_i + d) % N_DEV,),
                device_id_type=pl.DeviceIdType.MESH,
            )
        pl.semaphore_wait(barrier, 3)

        for c in kv_copies(0):
            c.start()
        q = jnp.dot(
            x_ref[...].astype(jnp.bfloat16),
            wq_ref[...].astype(jnp.bfloat16),
            preferred_element_type=jnp.float32,
        )
        qb[...] = (q * SCALE).astype(jnp.bfloat16)

        for h in range(HQ):
            if h + 1 < HQ:
                for c in kv_copies(h + 1):
                    c.start()
            for c in kv_copies(h):
                c.wait()
            slot = h % 2
            half, hh = divmod(h, HQ // 2)
            kb = kbuf[slot].astype(jnp.bfloat16)
            vb = vbuf[slot].astype(jnp.bfloat16)
            s = lax.dot_general(
                qb[:, h * DH:(h + 1) * DH],
                kb,
                (((1,), (1,)), ((), ())),
                preferred_element_type=jnp.float32,
            )
            p = jnp.exp(s)
            send_l[:, h:h + 1] = jnp.sum(p, axis=1, keepdims=True)
            o = jnp.dot(
                p.astype(jnp.bfloat16), vb, preferred_element_type=jnp.float32
            )
            send_o[half, :, pl.ds(hh * DH, DH)] = o.astype(jnp.bfloat16)
            if h == HQ // 2 - 1:
                for c in peer_copies(0, with_l=False):
                    c.start()

        for c in peer_copies(1, with_l=True):
            c.start()
        for c in peer_copies(0, with_l=False) + peer_copies(1, with_l=True):
            c.wait()

        l_tot = send_l[...] + recv_l[0] + recv_l[1] + recv_l[2]
        cols_out = []
        for hf in range(2):
            o_tot = (
                send_o[hf].astype(jnp.float32)
                + recv_o[0, hf].astype(jnp.float32)
                + recv_o[1, hf].astype(jnp.float32)
                + recv_o[2, hf].astype(jnp.float32)
            )
            for hq in range(HQ // 2):
                hg = hf * (HQ // 2) + hq
                cols_out.append(
                    (
                        o_tot[:, hq * DH:(hq + 1) * DH] / l_tot[:, hg:hg + 1]
                    ).astype(jnp.bfloat16)
                )
        attn = jnp.concatenate(cols_out, axis=1)
        out_ref[...] = jnp.dot(
            attn,
            wo_ref[...].astype(jnp.bfloat16),
            preferred_element_type=jnp.float32,
        )

    out = pl.pallas_call(
        body,
        out_shape=jax.ShapeDtypeStruct((SQ, D), jnp.float32),
        in_specs=[
            pl.BlockSpec(memory_space=pltpu.VMEM),
            pl.BlockSpec(memory_space=pltpu.VMEM),
            pl.BlockSpec(memory_space=pltpu.VMEM),
            pl.BlockSpec(memory_space=pl.MemorySpace.ANY),
            pl.BlockSpec(memory_space=pl.MemorySpace.ANY),
        ],
        out_specs=pl.BlockSpec(memory_space=pltpu.VMEM),
        scratch_shapes=[
            pltpu.VMEM((2, skv, DH), jnp.float32),
            pltpu.VMEM((2, skv, DH), jnp.float32),
            pltpu.VMEM((SQ, D), jnp.bfloat16),
            pltpu.VMEM((2, SQ, HALF), jnp.bfloat16),
            pltpu.VMEM((SQ, HQ), jnp.float32),
            pltpu.VMEM((N_DEV - 1, 2, SQ, HALF), jnp.bfloat16),
            pltpu.VMEM((N_DEV - 1, SQ, HQ), jnp.float32),
            pltpu.SemaphoreType.DMA((2, 2)),
            pltpu.SemaphoreType.DMA((N_DEV - 1, 2)),
            pltpu.SemaphoreType.DMA((N_DEV - 1,)),
            pltpu.SemaphoreType.DMA((N_DEV - 1, 2)),
            pltpu.SemaphoreType.DMA((N_DEV - 1,)),
        ],
        compiler_params=pltpu.CompilerParams(
            collective_id=0,
            vmem_limit_bytes=100 * 1024 * 1024,
        ),
    )(x2, Wq, Wo, k3, v3)
    return out.reshape(1, SQ, D)
